# baseline (speedup 1.0000x reference)
import functools
import jax
import jax.numpy as jnp
from jax import lax
from jax.experimental import pallas as pl
from jax.experimental.pallas import tpu as pltpu
from jax.experimental.pallas import tpu_sc as plsc

N = 10000
NP = 10240
E = 160000
NC = 2
NS = 16
NW = NC * NS
CH = 128
N_CHUNKS = E // CH
ITERS = (N_CHUNKS + NW - 1) // NW
ROWS_PER_TILE = NP // NS
NPF = 10112
RPTF = NPF // NS
BE = 2000
GE = E // BE

_sc_mesh = plsc.VectorSubcoreMesh(core_axis_name="c", subcore_axis_name="s")
_GATHER_DNUMS = lax.GatherDimensionNumbers(
    offset_dims=(), collapsed_slice_dims=(0,), start_index_map=(0,))


def _make_gather2(da, db):
    @functools.partial(
        pl.kernel,
        out_type=(
            jax.ShapeDtypeStruct((E, da), jnp.float32),
            jax.ShapeDtypeStruct((E, db), jnp.float32),
        ),
        mesh=_sc_mesh,
        scratch_types=[
            pltpu.VMEM((2, CH), jnp.int32),
            pltpu.VMEM((2, CH), jnp.int32),
            pltpu.VMEM((2, CH, da), jnp.float32),
            pltpu.VMEM((2, CH, db), jnp.float32),
            pltpu.SemaphoreType.DMA,
            pltpu.SemaphoreType.DMA,
            pltpu.SemaphoreType.DMA,
            pltpu.SemaphoreType.DMA,
        ],
    )
    def gather2(table_a, idx_a, table_b, idx_b, out_a, out_b,
                ia_v, ib_v, ra_v, rb_v, sa0, sb0, sa1, sb1):
        wid = lax.axis_index("s") * NC + lax.axis_index("c")
        sems = ((sa0, sb0), (sa1, sb1))

        def body(jo):
            for t in range(2):
                c = wid + (2 * jo + t) * NW

                @pl.when(c < N_CHUNKS)
                def _(c=c, t=t):
                    base = c * CH
                    pltpu.sync_copy(idx_a.at[pl.ds(base, CH)], ia_v.at[t])
                    pltpu.sync_copy(idx_b.at[pl.ds(base, CH)], ib_v.at[t])
                    pltpu.async_copy(table_a.at[ia_v.at[t]], ra_v.at[t],
                                     sems[t][0])
                    pltpu.async_copy(table_b.at[ib_v.at[t]], rb_v.at[t],
                                     sems[t][1])

            for t in range(2):
                c = wid + (2 * jo + t) * NW

                @pl.when(c < N_CHUNKS)
                def _(c=c, t=t):
                    base = c * CH
                    pltpu.make_async_copy(table_a.at[ia_v.at[t]], ra_v.at[t],
                                          sems[t][0]).wait()
                    pltpu.make_async_copy(table_b.at[ib_v.at[t]], rb_v.at[t],
                                          sems[t][1]).wait()
                    pltpu.sync_copy(ra_v.at[t], out_a.at[pl.ds(base, CH)])
                    pltpu.sync_copy(rb_v.at[t], out_b.at[pl.ds(base, CH)])

        pl.loop(0, (ITERS + 1) // 2)(body)

    return gather2


def _make_scatter_add(d):
    @functools.partial(
        pl.kernel,
        out_type=jax.ShapeDtypeStruct((NC, NP, d), jnp.float32),
        mesh=_sc_mesh,
        scratch_types=[
            pltpu.VMEM((2, CH), jnp.int32),
            pltpu.VMEM((2, CH, d), jnp.float32),
            pltpu.VMEM_SHARED((NP, d), jnp.float32),
            pltpu.SemaphoreType.DMA,
            pltpu.SemaphoreType.DMA,
        ],
    )
    def scatter_add(vals, idx, zeros, out, idx_v, vals_v, accum, sv0, sv1):
        cid = lax.axis_index("c")
        sid = lax.axis_index("s")
        wid = sid * NC + cid
        row0 = sid * ROWS_PER_TILE
        sems = (sv0, sv1)
        pltpu.sync_copy(zeros, accum.at[pl.ds(row0, ROWS_PER_TILE)])
        plsc.subcore_barrier()

        def body(jo):
            for t in range(2):
                c = wid + (2 * jo + t) * NW

                @pl.when(c < N_CHUNKS)
                def _(c=c, t=t):
                    base = c * CH
                    pltpu.sync_copy(idx.at[pl.ds(base, CH)], idx_v.at[t])
                    pltpu.async_copy(vals.at[pl.ds(base, CH)], vals_v.at[t],
                                     sems[t])

            for t in range(2):
                c = wid + (2 * jo + t) * NW

                @pl.when(c < N_CHUNKS)
                def _(c=c, t=t):
                    pltpu.make_async_copy(vals.at[pl.ds(c * CH, CH)],
                                          vals_v.at[t], sems[t]).wait()
                    pltpu.sync_copy(vals_v.at[t], accum.at[idx_v.at[t]],
                                    add=True)

        pl.loop(0, (ITERS + 1) // 2)(body)
        plsc.subcore_barrier()
        pltpu.sync_copy(accum.at[pl.ds(row0, ROWS_PER_TILE)],
                        out.at[cid, pl.ds(row0, ROWS_PER_TILE)])

    return scatter_add


def _pack_bf16_words(lo, hi):
    bl = jax.lax.bitcast_convert_type(lo, jnp.uint32)
    bh = jax.lax.bitcast_convert_type(hi, jnp.uint32)
    bl = (bl + jnp.uint32(0x8000)) >> 16
    bh = (bh + jnp.uint32(0x8000)) & jnp.uint32(0xFFFF0000)
    return jax.lax.bitcast_convert_type(bh | bl, jnp.float32)


def _unpack_bf16_words(w):
    b = jax.lax.bitcast_convert_type(w, jnp.uint32)
    lo = jax.lax.bitcast_convert_type(b << 16, jnp.float32)
    hi = jax.lax.bitcast_convert_type(b & jnp.uint32(0xFFFF0000), jnp.float32)
    return lo, hi


def _mmb(a, b):
    return jax.lax.dot_general(a.astype(jnp.bfloat16), b.astype(jnp.bfloat16),
                               (((1,), (0,)), ((), ())),
                               preferred_element_type=jnp.float32)


def _fused_layer1():
    CHF = 64
    NCF = E // CHF
    ITF = (NCF + NW - 1) // NW

    @functools.partial(
        pl.kernel,
        out_type=jax.ShapeDtypeStruct((NC, NPF, 128), jnp.float32),
        mesh=_sc_mesh,
        scratch_types=[
            pltpu.VMEM((2, CHF), jnp.int32),
            pltpu.VMEM((2, CHF), jnp.int32),
            pltpu.VMEM((2, CHF, 128), jnp.float32),
            pltpu.VMEM((2, CHF, 128), jnp.float32),
            pltpu.VMEM((CHF, 64), jnp.float32),
            pltpu.VMEM((CHF, 128), jnp.float32),
            pltpu.VMEM((64,), jnp.float32),
            pltpu.VMEM_SHARED((NPF, 128), jnp.float32),
            pltpu.SemaphoreType.DMA,
            pltpu.SemaphoreType.DMA,
            pltpu.SemaphoreType.DMA,
            pltpu.SemaphoreType.DMA,
        ],
    )
    def fused1(tsrc, idx_src, tdst, idx_dst, ew, attn_h, zeros, out,
               ia_v, ib_v, ra_v, rb_v, we_v, pay_v, attn_v, accum,
               sa0, sb0, sa1, sb1):
        cid = lax.axis_index("c")
        sid = lax.axis_index("s")
        wid = sid * NC + cid
        row0 = sid * RPTF
        sems = ((sa0, sb0), (sa1, sb1))
        pltpu.sync_copy(zeros, accum.at[pl.ds(row0, RPTF)])
        pltpu.sync_copy(attn_h, attn_v)

        def zrow(r):
            for cc in range(8):
                pay_v[r, pl.ds(cc * 16, 16)] = jnp.zeros((16,), jnp.float32)

        pl.loop(0, CHF)(zrow)
        plsc.subcore_barrier()

        def body(jo):
            for t in range(2):
                c = wid + (2 * jo + t) * NW

                @pl.when(c < NCF)
                def _(c=c, t=t):
                    base = c * CHF
                    pltpu.sync_copy(idx_src.at[pl.ds(base, CHF)], ia_v.at[t])
                    pltpu.sync_copy(idx_dst.at[pl.ds(base, CHF)], ib_v.at[t])
                    pltpu.async_copy(tsrc.at[ia_v.at[t]], ra_v.at[t], sems[t][0])
                    pltpu.async_copy(tdst.at[ib_v.at[t]], rb_v.at[t], sems[t][1])

            for t in range(2):
                c = wid + (2 * jo + t) * NW

                @pl.when(c < NCF)
                def _(c=c, t=t):
                    base = c * CHF
                    pltpu.sync_copy(ew.at[pl.ds(base, CHF)], we_v)
                    pltpu.make_async_copy(tsrc.at[ia_v.at[t]], ra_v.at[t],
                                          sems[t][0]).wait()
                    pltpu.make_async_copy(tdst.at[ib_v.at[t]], rb_v.at[t],
                                          sems[t][1]).wait()

                    def edge(e):
                        psum = jnp.zeros((16,), jnp.float32)
                        for d in range(4):
                            v = (ra_v[t, e, pl.ds(d * 16, 16)]
                                 + rb_v[t, e, pl.ds(d * 16, 16)]
                                 + we_v[e, pl.ds(d * 16, 16)])
                            v = jnp.where(v >= 0, v, 0.01 * v)
                            psum = psum + v * attn_v[pl.ds(d * 16, 16)]
                        s = psum
                        for k in (8, 4, 2, 1):
                            perm = (lax.iota(jnp.int32, 16) + k) & 15
                            s = s + lax.gather(
                                s, perm[:, None], _GATHER_DNUMS,
                                slice_sizes=(1,),
                                mode=lax.GatherScatterMode.PROMISE_IN_BOUNDS)
                        exv = jnp.exp(s)
                        for d in range(4):
                            pay_v[e, pl.ds(d * 16, 16)] = (
                                exv * ra_v[t, e, pl.ds(64 + d * 16, 16)])
                        is0 = lax.iota(jnp.int32, 16) == 0
                        pay_v[e, pl.ds(64, 16)] = jnp.where(is0, exv, 0.0)

                    pl.loop(0, CHF)(edge)
                    pltpu.sync_copy(pay_v, accum.at[ib_v.at[t]], add=True)

        pl.loop(0, (ITF + 1) // 2)(body)
        plsc.subcore_barrier()
        pltpu.sync_copy(accum.at[pl.ds(row0, RPTF)],
                        out.at[cid, pl.ds(row0, RPTF)])

    return fused1


def _leaky_relu(x):
    return jnp.where(x >= 0, x, 0.01 * x)


def _mm(a, b):
    return jax.lax.dot_general(a, b, (((1,), (0,)), ((), ())),
                               preferred_element_type=jnp.float32)


def _tc_node0(ff_ref, wfp, bfp, wni, wnj, wnode, tsrc, tdst):
    ff = ff_ref[...]
    h0 = _mm(ff, wfp[...]) + bfp[...]
    hi = _mm(h0, wni[...])
    hm = _mm(h0, wnode[...])
    z = jnp.zeros((NP, 32), jnp.float32)
    lo = jnp.concatenate([ff, hi, z], axis=1)
    hi2 = jnp.concatenate([hm, z, z], axis=1)
    tsrc[...] = _pack_bf16_words(lo, hi2)
    tdst[...] = jnp.concatenate([_mm(h0, wnj[...]), z, z], axis=1)


def _tc_edge0(g0a, g0b, ef_ref, wep, bep, wfij0, be0, attn0, wfij1, be1,
              wef2, bmat, rmask, tmask, v0, ew1):
    lo, hi2 = _unpack_bf16_words(g0a[...])
    gff = lo[:, 0:32]
    ghi = lo[:, 32:96]
    ghm = hi2[:, 0:64]
    ef = ef_ref[...]
    wc0 = _mm(wep[...], wfij0[...])
    bc0 = _mm(bep[...], wfij0[...]) + be0[...]
    ew0 = _mmb(ef, wc0) + bc0
    f0 = _leaky_relu(ghi + g0b[:, 0:64] + ew0)
    ex = jnp.exp(jnp.sum(f0 * attn0[...], axis=1, keepdims=True))
    ew1[...] = _mmb(f0, wfij1[...]) + be1[...]
    x = _mmb(ef, rmask[...]) * _mmb(gff, tmask[...])
    msg = _mmb(x, wef2[...]) + _mmb(gff, bmat[...])
    ci = lax.broadcasted_iota(jnp.int32, (BE, 32), 1)
    exdeg = jnp.where(ci == 0, ex, jnp.where(ci == 1, 1.0, 0.0))
    v0[...] = jnp.concatenate([ex * ghm, msg, exdeg], axis=1)


def _tc_node1(p0, bnn, wni, wnj, wnode, tsrc, tdst, ef_out):
    acc = p0[0] + p0[1]
    den = acc[:, 96:97]
    deg = acc[:, 97:98]
    h1 = acc[:, 0:64] / (den + 1e-16)
    ef_out[...] = acc[:, 64:96] / jnp.maximum(deg, 1.0) + bnn[...]
    hi = _mm(h1, wni[...])
    hm = _mm(h1, wnode[...])
    tsrc[...] = jnp.concatenate([hi, hm], axis=1)
    tdst[...] = jnp.concatenate([_mm(h1, wnj[...]),
                                 jnp.zeros((NP, 64), jnp.float32)], axis=1)


def _tc_edge1(g1a, g1b, ew1, attn1, v1):
    ghi = g1a[:, 0:64]
    ghm = g1a[:, 64:128]
    f1 = _leaky_relu(ghi + g1b[:, 0:64] + ew1[...])
    ex = jnp.exp(jnp.sum(f1 * attn1[...], axis=1, keepdims=True))
    ci = lax.broadcasted_iota(jnp.int32, (BE, 64), 1)
    exz = jnp.where(ci == 0, ex, 0.0)
    v1[...] = jnp.concatenate([ex * ghm, exz], axis=1)


def _tc_final(p1, ef_in, wgate, bgate, nf_out, gf_out):
    acc = p1[0, 0:N] + p1[1, 0:N]
    gf_nodes = acc[:, 0:64] / (acc[:, 64:65] + 1e-16)
    nf = jnp.concatenate([gf_nodes, ef_in[0:N],
                          jnp.zeros((N, 32), jnp.float32)], axis=1)
    g = _mm(nf, wgate[...]) + bgate[...]
    m = jnp.max(g)
    p = jnp.exp(g - m)
    gate = p / jnp.sum(p)
    nf_out[...] = nf
    gf_out[...] = jnp.sum(gate * nf, axis=0, keepdims=True)


def _full(shape):
    return pl.BlockSpec(shape, lambda *_: tuple(0 for _ in shape))


def kernel(face_features, edge_features, edge_index, W_fp, b_fp, W_ep, b_ep,
           W_ni_0, W_fij_0, W_nj_0, attn_0, be_0, W_node_0,
           W_ni_1, W_fij_1, W_nj_1, attn_1, be_1, W_node_1,
           W_ef, b_ef, b_nn, W_gate, b_gate):
    src = edge_index[0]
    dst = edge_index[1]
    f32 = jnp.float32
    ffp = jnp.pad(face_features, ((0, NP - N), (0, 0)))

    tsrc0, tdst0 = pl.pallas_call(
        _tc_node0,
        out_shape=(jax.ShapeDtypeStruct((NP, 128), f32),
                   jax.ShapeDtypeStruct((NP, 128), f32)),
    )(ffp, W_fp, b_fp.reshape(1, 64), W_ni_0, W_nj_0, W_node_0)

    g0a, g0b = _make_gather2(128, 128)(tsrc0, src, tdst0, dst)

    wef2 = W_ef.reshape(16, 32, 32).reshape(512, 32)
    bmat = b_ef.reshape(32, 32)
    v0_call = pl.pallas_call(
        _tc_edge0,
        grid=(GE,),
        in_specs=[
            pl.BlockSpec((BE, 128), lambda i: (i, 0)),
            pl.BlockSpec((BE, 128), lambda i: (i, 0)),
            pl.BlockSpec((BE, 16), lambda i: (i, 0)),
            _full((16, 64)), _full((1, 64)), _full((64, 64)), _full((1, 64)),
            _full((1, 64)), _full((64, 64)), _full((1, 64)),
            _full((512, 32)), _full((32, 32)),
            _full((16, 512)), _full((32, 512)),
        ],
        out_specs=(pl.BlockSpec((BE, 128), lambda i: (i, 0)),
                   pl.BlockSpec((BE, 64), lambda i: (i, 0))),
        out_shape=(jax.ShapeDtypeStruct((E, 128), f32),
                   jax.ShapeDtypeStruct((E, 64), f32)),
    )
    rmask = jnp.kron(jnp.eye(16, dtype=f32), jnp.ones((1, 32), f32))
    tmask = jnp.tile(jnp.eye(32, dtype=f32), (1, 16))
    v0, ew1 = v0_call(g0a, g0b, edge_features, W_ep, b_ep.reshape(1, 64),
                      W_fij_0, be_0.reshape(1, 64), attn_0.reshape(1, 64),
                      W_fij_1, be_1.reshape(1, 64), wef2, bmat, rmask, tmask)

    zeros = jnp.zeros((ROWS_PER_TILE, 128), f32)
    p0 = _make_scatter_add(128)(v0, dst, zeros)

    tsrc1, tdst1, ef_nodes = pl.pallas_call(
        _tc_node1,
        out_shape=(jax.ShapeDtypeStruct((NP, 128), f32),
                   jax.ShapeDtypeStruct((NP, 128), f32),
                   jax.ShapeDtypeStruct((NP, 32), f32)),
    )(p0, b_nn.reshape(1, 32), W_ni_1, W_nj_1, W_node_1)

    zeros_f = jnp.zeros((RPTF, 128), f32)
    p1 = _fused_layer1()(tsrc1, src, tdst1, dst, ew1, attn_1, zeros_f)

    nf, gf = pl.pallas_call(
        _tc_final,
        out_shape=(jax.ShapeDtypeStruct((N, 128), f32),
                   jax.ShapeDtypeStruct((1, 128), f32)),
    )(p1, ef_nodes, W_gate, b_gate.reshape(1, 1))

    return nf, gf


# speedup vs baseline: 1.0050x; 1.0050x over previous
import functools
import jax
import jax.numpy as jnp
from jax import lax
from jax.experimental import pallas as pl
from jax.experimental.pallas import tpu as pltpu
from jax.experimental.pallas import tpu_sc as plsc

N = 10000
NP = 10240
E = 160000
NC = 2
NS = 16
NW = NC * NS
CH = 128
N_CHUNKS = E // CH
ITERS = (N_CHUNKS + NW - 1) // NW
ROWS_PER_TILE = NP // NS
NPF = 10112
RPTF = NPF // NS
BE = 2000
GE = E // BE

_sc_mesh = plsc.VectorSubcoreMesh(core_axis_name="c", subcore_axis_name="s")
_GATHER_DNUMS = lax.GatherDimensionNumbers(
    offset_dims=(), collapsed_slice_dims=(0,), start_index_map=(0,))


def _make_gather2(da, db):
    @functools.partial(
        pl.kernel,
        out_type=(
            jax.ShapeDtypeStruct((E, da), jnp.float32),
            jax.ShapeDtypeStruct((E, db), jnp.float32),
        ),
        mesh=_sc_mesh,
        scratch_types=[
            pltpu.VMEM((ITERS * CH,), jnp.int32),
            pltpu.VMEM((ITERS * CH,), jnp.int32),
            pltpu.VMEM((2, CH, da), jnp.float32),
            pltpu.VMEM((2, CH, db), jnp.float32),
            pltpu.SemaphoreType.DMA,
            pltpu.SemaphoreType.DMA,
            pltpu.SemaphoreType.DMA,
            pltpu.SemaphoreType.DMA,
        ],
    )
    def gather2(table_a, idx_a, table_b, idx_b, out_a, out_b,
                ia_s, ib_s, ra_v, rb_v, sa0, sb0, sa1, sb1):
        wid = lax.axis_index("s") * NC + lax.axis_index("c")
        c0 = wid * ITERS
        cnt = jnp.minimum(ITERS, N_CHUNKS - c0)
        sems = ((sa0, sb0), (sa1, sb1))
        pltpu.sync_copy(idx_a.at[pl.ds(c0 * CH, ITERS * CH)], ia_s)
        pltpu.sync_copy(idx_b.at[pl.ds(c0 * CH, ITERS * CH)], ib_s)

        def body(jo):
            for t in range(2):
                k = 2 * jo + t

                @pl.when(k < cnt)
                def _(k=k, t=t):
                    pltpu.async_copy(table_a.at[ia_s.at[pl.ds(k * CH, CH)]],
                                     ra_v.at[t], sems[t][0])
                    pltpu.async_copy(table_b.at[ib_s.at[pl.ds(k * CH, CH)]],
                                     rb_v.at[t], sems[t][1])

            for t in range(2):
                k = 2 * jo + t

                @pl.when(k < cnt)
                def _(k=k, t=t):
                    base = (c0 + k) * CH
                    pltpu.make_async_copy(table_a.at[ia_s.at[pl.ds(k * CH, CH)]],
                                          ra_v.at[t], sems[t][0]).wait()
                    pltpu.make_async_copy(table_b.at[ib_s.at[pl.ds(k * CH, CH)]],
                                          rb_v.at[t], sems[t][1]).wait()
                    pltpu.sync_copy(ra_v.at[t], out_a.at[pl.ds(base, CH)])
                    pltpu.sync_copy(rb_v.at[t], out_b.at[pl.ds(base, CH)])

        pl.loop(0, (ITERS + 1) // 2)(body)

    return gather2


def _make_scatter_add(d):
    @functools.partial(
        pl.kernel,
        out_type=jax.ShapeDtypeStruct((NC, NP, d), jnp.float32),
        mesh=_sc_mesh,
        scratch_types=[
            pltpu.VMEM((ITERS * CH,), jnp.int32),
            pltpu.VMEM((2, CH, d), jnp.float32),
            pltpu.VMEM_SHARED((NP, d), jnp.float32),
            pltpu.SemaphoreType.DMA,
            pltpu.SemaphoreType.DMA,
        ],
    )
    def scatter_add(vals, idx, zeros, out, idx_s, vals_v, accum, sv0, sv1):
        cid = lax.axis_index("c")
        sid = lax.axis_index("s")
        wid = sid * NC + cid
        c0 = wid * ITERS
        cnt = jnp.minimum(ITERS, N_CHUNKS - c0)
        row0 = sid * ROWS_PER_TILE
        sems = (sv0, sv1)
        pltpu.sync_copy(zeros, accum.at[pl.ds(row0, ROWS_PER_TILE)])
        pltpu.sync_copy(idx.at[pl.ds(c0 * CH, ITERS * CH)], idx_s)
        plsc.subcore_barrier()

        def body(jo):
            for t in range(2):
                k = 2 * jo + t

                @pl.when(k < cnt)
                def _(k=k, t=t):
                    pltpu.async_copy(vals.at[pl.ds((c0 + k) * CH, CH)],
                                     vals_v.at[t], sems[t])

            for t in range(2):
                k = 2 * jo + t

                @pl.when(k < cnt)
                def _(k=k, t=t):
                    pltpu.make_async_copy(vals.at[pl.ds((c0 + k) * CH, CH)],
                                          vals_v.at[t], sems[t]).wait()
                    pltpu.sync_copy(vals_v.at[t],
                                    accum.at[idx_s.at[pl.ds(k * CH, CH)]],
                                    add=True)

        pl.loop(0, (ITERS + 1) // 2)(body)
        plsc.subcore_barrier()
        pltpu.sync_copy(accum.at[pl.ds(row0, ROWS_PER_TILE)],
                        out.at[cid, pl.ds(row0, ROWS_PER_TILE)])

    return scatter_add


def _pack_bf16_words(lo, hi):
    bl = jax.lax.bitcast_convert_type(lo, jnp.uint32)
    bh = jax.lax.bitcast_convert_type(hi, jnp.uint32)
    bl = (bl + jnp.uint32(0x8000)) >> 16
    bh = (bh + jnp.uint32(0x8000)) & jnp.uint32(0xFFFF0000)
    return jax.lax.bitcast_convert_type(bh | bl, jnp.float32)


def _unpack_bf16_words(w):
    b = jax.lax.bitcast_convert_type(w, jnp.uint32)
    lo = jax.lax.bitcast_convert_type(b << 16, jnp.float32)
    hi = jax.lax.bitcast_convert_type(b & jnp.uint32(0xFFFF0000), jnp.float32)
    return lo, hi


def _mmb(a, b):
    return jax.lax.dot_general(a.astype(jnp.bfloat16), b.astype(jnp.bfloat16),
                               (((1,), (0,)), ((), ())),
                               preferred_element_type=jnp.float32)


def _fused_layer1():
    CHF = 64
    NCF = E // CHF
    ITF = (NCF + NW - 1) // NW

    @functools.partial(
        pl.kernel,
        out_type=jax.ShapeDtypeStruct((NC, NPF, 128), jnp.float32),
        mesh=_sc_mesh,
        scratch_types=[
            pltpu.VMEM((2, CHF), jnp.int32),
            pltpu.VMEM((2, CHF), jnp.int32),
            pltpu.VMEM((2, CHF, 128), jnp.float32),
            pltpu.VMEM((2, CHF, 128), jnp.float32),
            pltpu.VMEM((CHF, 64), jnp.float32),
            pltpu.VMEM((CHF, 128), jnp.float32),
            pltpu.VMEM((64,), jnp.float32),
            pltpu.VMEM_SHARED((NPF, 128), jnp.float32),
            pltpu.SemaphoreType.DMA,
            pltpu.SemaphoreType.DMA,
            pltpu.SemaphoreType.DMA,
            pltpu.SemaphoreType.DMA,
        ],
    )
    def fused1(tsrc, idx_src, tdst, idx_dst, ew, attn_h, zeros, out,
               ia_v, ib_v, ra_v, rb_v, we_v, pay_v, attn_v, accum,
               sa0, sb0, sa1, sb1):
        cid = lax.axis_index("c")
        sid = lax.axis_index("s")
        wid = sid * NC + cid
        c0 = wid * ITF
        cnt = jnp.minimum(ITF, NCF - c0)
        row0 = sid * RPTF
        sems = ((sa0, sb0), (sa1, sb1))
        pltpu.sync_copy(zeros, accum.at[pl.ds(row0, RPTF)])
        pltpu.sync_copy(attn_h, attn_v)

        def zrow(r):
            for cc in range(8):
                pay_v[r, pl.ds(cc * 16, 16)] = jnp.zeros((16,), jnp.float32)

        pl.loop(0, CHF)(zrow)
        plsc.subcore_barrier()

        def body(jo):
            for t in range(2):
                k = 2 * jo + t

                @pl.when(k < cnt)
                def _(k=k, t=t):
                    base = (c0 + k) * CHF
                    pltpu.sync_copy(idx_src.at[pl.ds(base, CHF)], ia_v.at[t])
                    pltpu.sync_copy(idx_dst.at[pl.ds(base, CHF)], ib_v.at[t])
                    pltpu.async_copy(tsrc.at[ia_v.at[t]], ra_v.at[t],
                                     sems[t][0])
                    pltpu.async_copy(tdst.at[ib_v.at[t]], rb_v.at[t],
                                     sems[t][1])

            for t in range(2):
                k = 2 * jo + t

                @pl.when(k < cnt)
                def _(k=k, t=t):
                    base = (c0 + k) * CHF
                    pltpu.sync_copy(ew.at[pl.ds(base, CHF)], we_v)
                    pltpu.make_async_copy(tsrc.at[ia_v.at[t]], ra_v.at[t],
                                          sems[t][0]).wait()
                    pltpu.make_async_copy(tdst.at[ib_v.at[t]], rb_v.at[t],
                                          sems[t][1]).wait()

                    def edge(e):
                        psum = jnp.zeros((16,), jnp.float32)
                        for d in range(4):
                            v = (ra_v[t, e, pl.ds(d * 16, 16)]
                                 + rb_v[t, e, pl.ds(d * 16, 16)]
                                 + we_v[e, pl.ds(d * 16, 16)])
                            v = jnp.where(v >= 0, v, 0.01 * v)
                            psum = psum + v * attn_v[pl.ds(d * 16, 16)]
                        s = psum
                        for k in (8, 4, 2, 1):
                            perm = (lax.iota(jnp.int32, 16) + k) & 15
                            s = s + lax.gather(
                                s, perm[:, None], _GATHER_DNUMS,
                                slice_sizes=(1,),
                                mode=lax.GatherScatterMode.PROMISE_IN_BOUNDS)
                        exv = jnp.exp(s)
                        for d in range(4):
                            pay_v[e, pl.ds(d * 16, 16)] = (
                                exv * ra_v[t, e, pl.ds(64 + d * 16, 16)])
                        is0 = lax.iota(jnp.int32, 16) == 0
                        pay_v[e, pl.ds(64, 16)] = jnp.where(is0, exv, 0.0)

                    pl.loop(0, CHF)(edge)
                    pltpu.sync_copy(pay_v, accum.at[ib_v.at[t]], add=True)

        pl.loop(0, (ITF + 1) // 2)(body)
        plsc.subcore_barrier()
        pltpu.sync_copy(accum.at[pl.ds(row0, RPTF)],
                        out.at[cid, pl.ds(row0, RPTF)])

    return fused1


def _leaky_relu(x):
    return jnp.where(x >= 0, x, 0.01 * x)


def _mm(a, b):
    return jax.lax.dot_general(a, b, (((1,), (0,)), ((), ())),
                               preferred_element_type=jnp.float32)


def _tc_node0(ff_ref, wfp, bfp, wni, wnj, wnode, tsrc, tdst):
    ff = ff_ref[...]
    h0 = _mm(ff, wfp[...]) + bfp[...]
    hi = _mm(h0, wni[...])
    hm = _mm(h0, wnode[...])
    z = jnp.zeros((NP, 32), jnp.float32)
    lo = jnp.concatenate([ff, hi, z], axis=1)
    hi2 = jnp.concatenate([hm, z, z], axis=1)
    tsrc[...] = _pack_bf16_words(lo, hi2)
    tdst[...] = jnp.concatenate([_mm(h0, wnj[...]), z, z], axis=1)


def _tc_edge0(g0a, g0b, ef_ref, wep, bep, wfij0, be0, attn0, wfij1, be1,
              wef2, bmat, rmask, tmask, v0, ew1):
    lo, hi2 = _unpack_bf16_words(g0a[...])
    gff = lo[:, 0:32]
    ghi = lo[:, 32:96]
    ghm = hi2[:, 0:64]
    ef = ef_ref[...]
    wc0 = _mm(wep[...], wfij0[...])
    bc0 = _mm(bep[...], wfij0[...]) + be0[...]
    ew0 = _mmb(ef, wc0) + bc0
    f0 = _leaky_relu(ghi + g0b[:, 0:64] + ew0)
    ex = jnp.exp(jnp.sum(f0 * attn0[...], axis=1, keepdims=True))
    ew1[...] = _mmb(f0, wfij1[...]) + be1[...]
    x = _mmb(ef, rmask[...]) * _mmb(gff, tmask[...])
    msg = _mmb(x, wef2[...]) + _mmb(gff, bmat[...])
    ci = lax.broadcasted_iota(jnp.int32, (BE, 32), 1)
    exdeg = jnp.where(ci == 0, ex, jnp.where(ci == 1, 1.0, 0.0))
    v0[...] = jnp.concatenate([ex * ghm, msg, exdeg], axis=1)


def _tc_node1(p0, bnn, wni, wnj, wnode, tsrc, tdst, ef_out):
    acc = p0[0] + p0[1]
    den = acc[:, 96:97]
    deg = acc[:, 97:98]
    h1 = acc[:, 0:64] / (den + 1e-16)
    ef_out[...] = acc[:, 64:96] / jnp.maximum(deg, 1.0) + bnn[...]
    hi = _mm(h1, wni[...])
    hm = _mm(h1, wnode[...])
    tsrc[...] = jnp.concatenate([hi, hm], axis=1)
    tdst[...] = jnp.concatenate([_mm(h1, wnj[...]),
                                 jnp.zeros((NP, 64), jnp.float32)], axis=1)


def _tc_edge1(g1a, g1b, ew1, attn1, v1):
    ghi = g1a[:, 0:64]
    ghm = g1a[:, 64:128]
    f1 = _leaky_relu(ghi + g1b[:, 0:64] + ew1[...])
    ex = jnp.exp(jnp.sum(f1 * attn1[...], axis=1, keepdims=True))
    ci = lax.broadcasted_iota(jnp.int32, (BE, 64), 1)
    exz = jnp.where(ci == 0, ex, 0.0)
    v1[...] = jnp.concatenate([ex * ghm, exz], axis=1)


def _tc_final(p1, ef_in, wgate, bgate, nf_out, gf_out):
    acc = p1[0, 0:N] + p1[1, 0:N]
    gf_nodes = acc[:, 0:64] / (acc[:, 64:65] + 1e-16)
    nf = jnp.concatenate([gf_nodes, ef_in[0:N],
                          jnp.zeros((N, 32), jnp.float32)], axis=1)
    g = _mm(nf, wgate[...]) + bgate[...]
    m = jnp.max(g)
    p = jnp.exp(g - m)
    gate = p / jnp.sum(p)
    nf_out[...] = nf
    gf_out[...] = jnp.sum(gate * nf, axis=0, keepdims=True)


def _full(shape):
    return pl.BlockSpec(shape, lambda *_: tuple(0 for _ in shape))


def kernel(face_features, edge_features, edge_index, W_fp, b_fp, W_ep, b_ep,
           W_ni_0, W_fij_0, W_nj_0, attn_0, be_0, W_node_0,
           W_ni_1, W_fij_1, W_nj_1, attn_1, be_1, W_node_1,
           W_ef, b_ef, b_nn, W_gate, b_gate):
    pade = NW * ITERS * CH
    src = jnp.pad(edge_index[0], (0, pade - E))
    dst = jnp.pad(edge_index[1], (0, pade - E))
    f32 = jnp.float32
    ffp = jnp.pad(face_features, ((0, NP - N), (0, 0)))

    tsrc0, tdst0 = pl.pallas_call(
        _tc_node0,
        out_shape=(jax.ShapeDtypeStruct((NP, 128), f32),
                   jax.ShapeDtypeStruct((NP, 128), f32)),
    )(ffp, W_fp, b_fp.reshape(1, 64), W_ni_0, W_nj_0, W_node_0)

    g0a, g0b = _make_gather2(128, 128)(tsrc0, src, tdst0, dst)

    wef2 = W_ef.reshape(16, 32, 32).reshape(512, 32)
    bmat = b_ef.reshape(32, 32)
    v0_call = pl.pallas_call(
        _tc_edge0,
        grid=(GE,),
        in_specs=[
            pl.BlockSpec((BE, 128), lambda i: (i, 0)),
            pl.BlockSpec((BE, 128), lambda i: (i, 0)),
            pl.BlockSpec((BE, 16), lambda i: (i, 0)),
            _full((16, 64)), _full((1, 64)), _full((64, 64)), _full((1, 64)),
            _full((1, 64)), _full((64, 64)), _full((1, 64)),
            _full((512, 32)), _full((32, 32)),
            _full((16, 512)), _full((32, 512)),
        ],
        out_specs=(pl.BlockSpec((BE, 128), lambda i: (i, 0)),
                   pl.BlockSpec((BE, 64), lambda i: (i, 0))),
        out_shape=(jax.ShapeDtypeStruct((E, 128), f32),
                   jax.ShapeDtypeStruct((E, 64), f32)),
    )
    rmask = jnp.kron(jnp.eye(16, dtype=f32), jnp.ones((1, 32), f32))
    tmask = jnp.tile(jnp.eye(32, dtype=f32), (1, 16))
    v0, ew1 = v0_call(g0a, g0b, edge_features, W_ep, b_ep.reshape(1, 64),
                      W_fij_0, be_0.reshape(1, 64), attn_0.reshape(1, 64),
                      W_fij_1, be_1.reshape(1, 64), wef2, bmat, rmask, tmask)

    zeros = jnp.zeros((ROWS_PER_TILE, 128), f32)
    p0 = _make_scatter_add(128)(v0, dst, zeros)

    tsrc1, tdst1, ef_nodes = pl.pallas_call(
        _tc_node1,
        out_shape=(jax.ShapeDtypeStruct((NP, 128), f32),
                   jax.ShapeDtypeStruct((NP, 128), f32),
                   jax.ShapeDtypeStruct((NP, 32), f32)),
    )(p0, b_nn.reshape(1, 32), W_ni_1, W_nj_1, W_node_1)

    zeros_f = jnp.zeros((RPTF, 128), f32)
    p1 = _fused_layer1()(tsrc1, src, tdst1, dst, ew1, attn_1, zeros_f)

    nf, gf = pl.pallas_call(
        _tc_final,
        out_shape=(jax.ShapeDtypeStruct((N, 128), f32),
                   jax.ShapeDtypeStruct((1, 128), f32)),
    )(p1, ef_nodes, W_gate, b_gate.reshape(1, 1))

    return nf, gf
